# Initial kernel scaffold; baseline (speedup 1.0000x reference)
import jax
import jax.numpy as jnp
from jax.experimental import pallas as pl


def kernel(x, edge_ast, edge_cfg, W_ast, b_ast, W_cfg, b_cfg, W_ih, W_hh, b_ih, b_hh):
    raise NotImplementedError("write your pallas kernel here")


# speedup vs baseline: 5.4137x; 5.4137x over previous
import functools

import jax
import jax.numpy as jnp
from jax import lax
from jax.experimental import pallas as pl
from jax.experimental.pallas import tpu as pltpu
from jax.experimental.pallas import tpu_sc as plsc

N = 10000
H = 128
E = 320000
NC = 2
NS = 16
CH = 128
NPAD = 10240
RPT = NPAD // (NC * NS)
RPT16 = NPAD // NS
EPAD = 327680
SCH = EPAD // (NC * NS * CH)
DCH = EPAD // (NS * CH)
BR = 1024

_mesh = plsc.VectorSubcoreMesh(core_axis_name="c", subcore_axis_name="s")


@functools.partial(
    pl.kernel,
    out_type=jax.ShapeDtypeStruct((NC * NPAD,), jnp.float32),
    mesh=_mesh,
    scratch_types=[
        pltpu.VMEM((DCH, CH), jnp.int32),
        pltpu.VMEM((CH,), jnp.float32),
        pltpu.VMEM_SHARED((NPAD,), jnp.float32),
    ],
)
def _sc_degree(dst_hbm, ones_hbm, zeros1_hbm, out_hbm, dst_v, ones_v, sdeg):
    c = lax.axis_index("c")
    s = lax.axis_index("s")
    pltpu.sync_copy(ones_hbm, ones_v)
    pltpu.sync_copy(zeros1_hbm, sdeg.at[pl.ds(s * RPT16, RPT16)])
    plsc.subcore_barrier()
    pltpu.sync_copy(dst_hbm.at[c, s], dst_v)

    def body(j, _):
        pltpu.sync_copy(ones_v, sdeg.at[dst_v.at[j]], add=True)
        return ()

    lax.fori_loop(0, DCH, body, (), unroll=False)
    plsc.subcore_barrier()
    pltpu.sync_copy(sdeg.at[pl.ds(s * RPT16, RPT16)],
                    out_hbm.at[pl.ds(c * NPAD + s * RPT16, RPT16)])


@functools.partial(
    pl.kernel,
    out_type=jax.ShapeDtypeStruct((NC * NPAD, H), jnp.float32),
    mesh=_mesh,
    scratch_types=[
        pltpu.VMEM((SCH, CH), jnp.int32),
        pltpu.VMEM((SCH, CH), jnp.int32),
        pltpu.VMEM((CH, H), jnp.float32),
        pltpu.VMEM_SHARED((NPAD, H), jnp.float32),
        pltpu.SemaphoreType.DMA,
    ],
)
def _sc_scatter(g_hbm, src_hbm, dst_hbm, zeros_hbm, out_hbm,
                src_v, dst_v, rows_v, ush, sem):
    c = lax.axis_index("c")
    s = lax.axis_index("s")
    pltpu.sync_copy(zeros_hbm, ush.at[pl.ds(s * RPT16, RPT16)])
    plsc.subcore_barrier()
    pltpu.sync_copy(src_hbm.at[c, s], src_v)
    pltpu.sync_copy(dst_hbm.at[c, s], dst_v)

    def body(j, _):
        pltpu.async_copy(g_hbm.at[src_v.at[j]], rows_v, sem).wait()
        pltpu.sync_copy(rows_v, ush.at[dst_v.at[j]], add=True)
        return ()

    lax.fori_loop(0, SCH, body, (), unroll=False)
    plsc.subcore_barrier()
    pltpu.sync_copy(ush.at[pl.ds(s * RPT16, RPT16)],
                    out_hbm.at[pl.ds(c * NPAD + s * RPT16, RPT16)])


def _pre_body(h_ref, w_ref, dega_ref, degc_ref, ga_ref, gc_ref):
    i = pl.program_id(0)
    rows = i * BR + lax.broadcasted_iota(jnp.int32, (BR, 1), 0)
    mask = (rows < N).astype(jnp.float32)
    dis_a = lax.rsqrt(dega_ref[...] + 1.0) * mask
    dis_c = lax.rsqrt(degc_ref[...] + 1.0) * mask
    hw = jnp.dot(h_ref[...], w_ref[...], preferred_element_type=jnp.float32)
    ga_ref[...] = hw[:, :H] * dis_a
    gc_ref[...] = hw[:, H:] * dis_c


def _tc_pre(h, wcat, deg_a, deg_c):
    return pl.pallas_call(
        _pre_body,
        grid=(NPAD // BR,),
        in_specs=[
            pl.BlockSpec((BR, H), lambda i: (i, 0)),
            pl.BlockSpec((H, 2 * H), lambda i: (0, 0)),
            pl.BlockSpec((BR, 1), lambda i: (i, 0)),
            pl.BlockSpec((BR, 1), lambda i: (i, 0)),
        ],
        out_specs=[
            pl.BlockSpec((BR, H), lambda i: (i, 0)),
            pl.BlockSpec((BR, H), lambda i: (i, 0)),
        ],
        out_shape=[
            jax.ShapeDtypeStruct((NPAD, H), jnp.float32),
            jax.ShapeDtypeStruct((NPAD, H), jnp.float32),
        ],
    )(h, wcat, deg_a, deg_c)


def _post_body(ua_ref, uc_ref, ga_ref, gc_ref, dega_ref, degc_ref, h_ref,
               wih_ref, whh_ref, bih_ref, bhh_ref, ba_ref, bc_ref, out_ref):
    i = pl.program_id(0)
    rows = i * BR + lax.broadcasted_iota(jnp.int32, (BR, 1), 0)
    mask = (rows < N).astype(jnp.float32)
    dis_a = lax.rsqrt(dega_ref[...] + 1.0) * mask
    dis_c = lax.rsqrt(degc_ref[...] + 1.0) * mask
    a = ((ua_ref[0] + ua_ref[1] + ga_ref[...]) * dis_a + ba_ref[...]
         + (uc_ref[0] + uc_ref[1] + gc_ref[...]) * dis_c + bc_ref[...])
    h = h_ref[...]
    gi = jnp.dot(a, wih_ref[...], preferred_element_type=jnp.float32) + bih_ref[...]
    gh = jnp.dot(h, whh_ref[...], preferred_element_type=jnp.float32) + bhh_ref[...]
    r = jax.nn.sigmoid(gi[:, :H] + gh[:, :H])
    z = jax.nn.sigmoid(gi[:, H:2 * H] + gh[:, H:2 * H])
    n = jnp.tanh(gi[:, 2 * H:] + r * gh[:, 2 * H:])
    out_ref[...] = (1.0 - z) * n + z * h


def _tc_post(ua, uc, g_a, g_c, deg_a, deg_c, h, wiht, whht, bih, bhh, ba, bc):
    blk = lambda i: (i, 0)
    return pl.pallas_call(
        _post_body,
        grid=(NPAD // BR,),
        in_specs=[
            pl.BlockSpec((2, BR, H), lambda i: (0, i, 0)),
            pl.BlockSpec((2, BR, H), lambda i: (0, i, 0)),
            pl.BlockSpec((BR, H), blk),
            pl.BlockSpec((BR, H), blk),
            pl.BlockSpec((BR, 1), blk),
            pl.BlockSpec((BR, 1), blk),
            pl.BlockSpec((BR, H), blk),
            pl.BlockSpec((H, 3 * H), lambda i: (0, 0)),
            pl.BlockSpec((H, 3 * H), lambda i: (0, 0)),
            pl.BlockSpec((1, 3 * H), lambda i: (0, 0)),
            pl.BlockSpec((1, 3 * H), lambda i: (0, 0)),
            pl.BlockSpec((1, H), lambda i: (0, 0)),
            pl.BlockSpec((1, H), lambda i: (0, 0)),
        ],
        out_specs=pl.BlockSpec((BR, H), blk),
        out_shape=jax.ShapeDtypeStruct((NPAD, H), jnp.float32),
    )(ua, uc, g_a, g_c, deg_a, deg_c, h, wiht, whht, bih, bhh, ba, bc)


def _pad_edges(e):
    pad = jnp.full((EPAD - E,), N, dtype=jnp.int32)
    src = jnp.concatenate([e[0].astype(jnp.int32), pad])
    dst = jnp.concatenate([e[1].astype(jnp.int32), pad])
    return src, dst


def kernel(x, edge_ast, edge_cfg, W_ast, b_ast, W_cfg, b_cfg,
           W_ih, W_hh, b_ih, b_hh):
    src_a, dst_a = _pad_edges(edge_ast)
    src_c, dst_c = _pad_edges(edge_cfg)

    dst_deg = jnp.stack([dst_a.reshape(NS, DCH, CH), dst_c.reshape(NS, DCH, CH)])
    ones = jnp.ones((CH,), jnp.float32)
    zeros1 = jnp.zeros((RPT16,), jnp.float32)
    deg2 = _sc_degree(dst_deg, ones, zeros1)
    deg_a = deg2[:NPAD].reshape(NPAD, 1)
    deg_c = deg2[NPAD:].reshape(NPAD, 1)

    src_a4 = src_a.reshape(NC, NS, SCH, CH)
    dst_a4 = dst_a.reshape(NC, NS, SCH, CH)
    src_c4 = src_c.reshape(NC, NS, SCH, CH)
    dst_c4 = dst_c.reshape(NC, NS, SCH, CH)
    zeros2 = jnp.zeros((RPT16, H), jnp.float32)

    h = jnp.pad(x, ((0, NPAD - N), (0, 0)))
    wcat = jnp.concatenate([W_ast, W_cfg], axis=1)
    wiht = W_ih.T
    whht = W_hh.T
    bih = b_ih.reshape(1, 3 * H)
    bhh = b_hh.reshape(1, 3 * H)
    ba = b_ast.reshape(1, H)
    bc = b_cfg.reshape(1, H)

    for _ in range(3):
        g_a, g_c = _tc_pre(h, wcat, deg_a, deg_c)
        u_a = _sc_scatter(g_a, src_a4, dst_a4, zeros2)
        u_c = _sc_scatter(g_c, src_c4, dst_c4, zeros2)
        h = _tc_post(u_a.reshape(NC, NPAD, H), u_c.reshape(NC, NPAD, H),
                     g_a, g_c, deg_a, deg_c, h, wiht, whht, bih, bhh, ba, bc)
    return h[:N]


# speedup vs baseline: 9.4565x; 1.7468x over previous
import functools

import jax
import jax.numpy as jnp
from jax import lax
from jax.experimental import pallas as pl
from jax.experimental.pallas import tpu as pltpu
from jax.experimental.pallas import tpu_sc as plsc

N = 10000
H = 128
E = 320000
NC = 2
NS = 16
CH = 128
NPAD = 10240
RPT16 = NPAD // NS
EPAD = 327680
DCH = EPAD // (NS * CH)
PB = 40
BR = 1024

_mesh = plsc.VectorSubcoreMesh(core_axis_name="c", subcore_axis_name="s")


@functools.partial(
    pl.kernel,
    out_type=jax.ShapeDtypeStruct((NC * NPAD,), jnp.float32),
    mesh=_mesh,
    scratch_types=[
        pltpu.VMEM((DCH, CH), jnp.int32),
        pltpu.VMEM((CH,), jnp.float32),
        pltpu.VMEM_SHARED((NPAD,), jnp.float32),
    ],
)
def _sc_degree(dst_hbm, ones_hbm, zeros1_hbm, out_hbm, dst_v, ones_v, sdeg):
    c = lax.axis_index("c")
    s = lax.axis_index("s")
    pltpu.sync_copy(ones_hbm, ones_v)
    pltpu.sync_copy(zeros1_hbm, sdeg.at[pl.ds(s * RPT16, RPT16)])
    plsc.subcore_barrier()
    pltpu.sync_copy(dst_hbm.at[c, s], dst_v)

    def body(j, _):
        pltpu.sync_copy(ones_v, sdeg.at[dst_v.at[j]], add=True)
        return ()

    lax.fori_loop(0, DCH, body, (), unroll=False)
    plsc.subcore_barrier()
    pltpu.sync_copy(sdeg.at[pl.ds(s * RPT16, RPT16)],
                    out_hbm.at[pl.ds(c * NPAD + s * RPT16, RPT16)])


@functools.partial(
    pl.kernel,
    out_type=jax.ShapeDtypeStruct((NC * NPAD, H), jnp.float32),
    mesh=_mesh,
    scratch_types=[
        pltpu.VMEM((PB, CH), jnp.int32),
        pltpu.VMEM((PB, CH), jnp.int32),
        pltpu.VMEM((CH, H), jnp.float32),
        pltpu.VMEM((CH, H), jnp.float32),
        pltpu.VMEM_SHARED((NPAD, H), jnp.float32),
        pltpu.SemaphoreType.DMA,
    ],
)
def _sc_scatter(g_hbm, src_hbm, dst_hbm, zeros_hbm, out_hbm,
                src_v, dst_v, rows0, rows1, ush, sem):
    c = lax.axis_index("c")
    s = lax.axis_index("s")
    pltpu.sync_copy(zeros_hbm, ush.at[pl.ds(s * RPT16, RPT16)])
    plsc.subcore_barrier()

    def gather(j, buf):
        return pltpu.make_async_copy(g_hbm.at[src_v.at[j]], buf, sem)

    def scat(j, buf):
        pltpu.sync_copy(buf, ush.at[dst_v.at[j]], add=True)

    for b in range(DCH // PB):
        pltpu.sync_copy(src_hbm.at[c, s, pl.ds(b * PB, PB)], src_v)
        pltpu.sync_copy(dst_hbm.at[c, s, pl.ds(b * PB, PB)], dst_v)
        gather(0, rows0).start()
        gather(1, rows1).start()

        def body(p, _):
            j = 2 * p
            gather(j, rows0).wait()
            scat(j, rows0)
            gather(j + 2, rows0).start()
            gather(j + 1, rows1).wait()
            scat(j + 1, rows1)
            gather(j + 3, rows1).start()
            return ()

        lax.fori_loop(0, PB // 2 - 1, body, (), unroll=False)
        gather(PB - 2, rows0).wait()
        scat(PB - 2, rows0)
        gather(PB - 1, rows1).wait()
        scat(PB - 1, rows1)
    plsc.subcore_barrier()
    pltpu.sync_copy(ush.at[pl.ds(s * RPT16, RPT16)],
                    out_hbm.at[pl.ds(c * NPAD + s * RPT16, RPT16)])


def _pre_body(h_ref, w_ref, deg_ref, g_ref):
    j = pl.program_id(0)
    i = j % (NPAD // BR)
    rows = i * BR + lax.broadcasted_iota(jnp.int32, (BR, 1), 0)
    mask = (rows < N).astype(jnp.float32)
    dis = lax.rsqrt(deg_ref[...] + 1.0) * mask
    g_ref[...] = jnp.dot(h_ref[...], w_ref[...],
                         preferred_element_type=jnp.float32) * dis


def _tc_pre(h, wcat, deg):
    nb = NPAD // BR
    return pl.pallas_call(
        _pre_body,
        grid=(2 * nb,),
        in_specs=[
            pl.BlockSpec((BR, H), lambda j: (j % nb, 0)),
            pl.BlockSpec((H, H), lambda j: (0, j // nb)),
            pl.BlockSpec((BR, 1), lambda j: (j, 0)),
        ],
        out_specs=pl.BlockSpec((BR, H), lambda j: (j, 0)),
        out_shape=jax.ShapeDtypeStruct((2 * NPAD, H), jnp.float32),
    )(h, wcat, deg)


def _post_body(u_ref, g_ref, deg_ref, h_ref,
               wih_ref, whh_ref, bih_ref, bhh_ref, ba_ref, bc_ref, out_ref):
    i = pl.program_id(0)
    rows = i * BR + lax.broadcasted_iota(jnp.int32, (BR, 1), 0)
    mask = (rows < N).astype(jnp.float32)
    dis_a = lax.rsqrt(deg_ref[0] + 1.0) * mask
    dis_c = lax.rsqrt(deg_ref[1] + 1.0) * mask
    a = ((u_ref[0] + g_ref[0]) * dis_a + ba_ref[...]
         + (u_ref[1] + g_ref[1]) * dis_c + bc_ref[...])
    h = h_ref[...]
    gi = jnp.dot(a, wih_ref[...], preferred_element_type=jnp.float32) + bih_ref[...]
    gh = jnp.dot(h, whh_ref[...], preferred_element_type=jnp.float32) + bhh_ref[...]
    r = jax.nn.sigmoid(gi[:, :H] + gh[:, :H])
    z = jax.nn.sigmoid(gi[:, H:2 * H] + gh[:, H:2 * H])
    n = jnp.tanh(gi[:, 2 * H:] + r * gh[:, 2 * H:])
    out_ref[...] = (1.0 - z) * n + z * h


def _tc_post(u3, g3, deg3, h, wiht, whht, bih, bhh, ba, bc):
    blk = lambda i: (i, 0)
    return pl.pallas_call(
        _post_body,
        grid=(NPAD // BR,),
        in_specs=[
            pl.BlockSpec((2, BR, H), lambda i: (0, i, 0)),
            pl.BlockSpec((2, BR, H), lambda i: (0, i, 0)),
            pl.BlockSpec((2, BR, 1), lambda i: (0, i, 0)),
            pl.BlockSpec((BR, H), blk),
            pl.BlockSpec((H, 3 * H), lambda i: (0, 0)),
            pl.BlockSpec((H, 3 * H), lambda i: (0, 0)),
            pl.BlockSpec((1, 3 * H), lambda i: (0, 0)),
            pl.BlockSpec((1, 3 * H), lambda i: (0, 0)),
            pl.BlockSpec((1, H), lambda i: (0, 0)),
            pl.BlockSpec((1, H), lambda i: (0, 0)),
        ],
        out_specs=pl.BlockSpec((BR, H), blk),
        out_shape=jax.ShapeDtypeStruct((NPAD, H), jnp.float32),
    )(u3, g3, deg3, h, wiht, whht, bih, bhh, ba, bc)


def kernel(x, edge_ast, edge_cfg, W_ast, b_ast, W_cfg, b_cfg,
           W_ih, W_hh, b_ih, b_hh):
    pad = jnp.full((EPAD - E,), N, dtype=jnp.int32)
    src_a = jnp.concatenate([edge_ast[0].astype(jnp.int32), pad])
    dst_a = jnp.concatenate([edge_ast[1].astype(jnp.int32), pad])
    src_c = jnp.concatenate([edge_cfg[0].astype(jnp.int32), pad]) + NPAD
    dst_c = jnp.concatenate([edge_cfg[1].astype(jnp.int32), pad])

    dst4 = jnp.stack([dst_a.reshape(NS, DCH, CH), dst_c.reshape(NS, DCH, CH)])
    src4 = jnp.stack([src_a.reshape(NS, DCH, CH), src_c.reshape(NS, DCH, CH)])
    ones = jnp.ones((CH,), jnp.float32)
    zeros1 = jnp.zeros((RPT16,), jnp.float32)
    zeros2 = jnp.zeros((RPT16, H), jnp.float32)

    deg2 = _sc_degree(dst4, ones, zeros1)
    deg = deg2.reshape(2 * NPAD, 1)
    deg3 = deg2.reshape(2, NPAD, 1)

    h = jnp.pad(x, ((0, NPAD - N), (0, 0)))
    wcat = jnp.concatenate([W_ast, W_cfg], axis=1)
    wiht = W_ih.T
    whht = W_hh.T
    bih = b_ih.reshape(1, 3 * H)
    bhh = b_hh.reshape(1, 3 * H)
    ba = b_ast.reshape(1, H)
    bc = b_cfg.reshape(1, H)

    for _ in range(3):
        g = _tc_pre(h, wcat, deg)
        u = _sc_scatter(g, src4, dst4, zeros2)
        h = _tc_post(u.reshape(2, NPAD, H), g.reshape(2, NPAD, H), deg3,
                     h, wiht, whht, bih, bhh, ba, bc)
    return h[:N]
